# baseline (speedup 1.0000x reference)
import jax
import jax.numpy as jnp
from jax.experimental import pallas as pl
from jax.experimental.pallas import tpu as pltpu

N = 4096
IN_FEAT = 64
OUT_FEAT = 64
BM = 512
H = N // 2
TPH = H // BM


def _kern(feat_ref, w_ref, adj_ref, out_ref, x_ref):
    i = pl.program_id(0)

    @pl.when(i == 0)
    def _():
        x_ref[...] = jnp.dot(
            feat_ref[...], w_ref[...], preferred_element_type=jnp.float32
        )

    y = jnp.dot(adj_ref[...], x_ref[...], preferred_element_type=jnp.float32)

    @pl.when(i < TPH)
    def _():
        out_ref[pl.ds(i * BM, BM), :OUT_FEAT] = y

    @pl.when(i >= TPH)
    def _():
        out_ref[pl.ds((i - TPH) * BM, BM), OUT_FEAT:] = y


@jax.jit
def kernel(feat, adj, weight2):
    grid = (N // BM,)
    outw = pl.pallas_call(
        _kern,
        grid=grid,
        in_specs=[
            pl.BlockSpec((N, IN_FEAT), lambda i: (0, 0)),
            pl.BlockSpec((IN_FEAT, OUT_FEAT), lambda i: (0, 0)),
            pl.BlockSpec((BM, N), lambda i: (i, 0)),
        ],
        out_specs=pl.BlockSpec((H, 2 * OUT_FEAT), lambda i: (0, 0)),
        out_shape=jax.ShapeDtypeStruct((H, 2 * OUT_FEAT), jnp.float32),
        scratch_shapes=[pltpu.VMEM((N, OUT_FEAT), jnp.float32)],
    )(feat, weight2, adj)
    return jnp.concatenate([outw[:, :OUT_FEAT], outw[:, OUT_FEAT:]], axis=0)


# speedup vs baseline: 1.0027x; 1.0027x over previous
import jax
import jax.numpy as jnp
from jax.experimental import pallas as pl
from jax.experimental.pallas import tpu as pltpu

N = 4096
IN_FEAT = 64
OUT_FEAT = 64
BM = 512
H = N // 2
TPH = H // BM


def _kern(featw_ref, w_ref, adj_ref, out_ref, x_ref):
    i = pl.program_id(0)

    @pl.when(i == 0)
    def _():
        w = w_ref[...]
        x_ref[:H, :] = jnp.dot(
            featw_ref[:, :IN_FEAT], w, preferred_element_type=jnp.float32
        )
        x_ref[H:, :] = jnp.dot(
            featw_ref[:, IN_FEAT:], w, preferred_element_type=jnp.float32
        )

    y = jnp.dot(adj_ref[...], x_ref[...], preferred_element_type=jnp.float32)

    @pl.when(i < TPH)
    def _():
        out_ref[pl.ds(i * BM, BM), :OUT_FEAT] = y

    @pl.when(i >= TPH)
    def _():
        out_ref[pl.ds((i - TPH) * BM, BM), OUT_FEAT:] = y


@jax.jit
def kernel(feat, adj, weight2):
    featw = jnp.concatenate([feat[:H], feat[H:]], axis=1)
    grid = (N // BM,)
    outw = pl.pallas_call(
        _kern,
        grid=grid,
        in_specs=[
            pl.BlockSpec((H, 2 * IN_FEAT), lambda i: (0, 0)),
            pl.BlockSpec((IN_FEAT, OUT_FEAT), lambda i: (0, 0)),
            pl.BlockSpec((BM, N), lambda i: (i, 0)),
        ],
        out_specs=pl.BlockSpec((H, 2 * OUT_FEAT), lambda i: (0, 0)),
        out_shape=jax.ShapeDtypeStruct((H, 2 * OUT_FEAT), jnp.float32),
        scratch_shapes=[pltpu.VMEM((N, OUT_FEAT), jnp.float32)],
    )(featw, weight2, adj)
    return jnp.concatenate([outw[:, :OUT_FEAT], outw[:, OUT_FEAT:]], axis=0)


# speedup vs baseline: 1.0166x; 1.0138x over previous
import jax
import jax.numpy as jnp
from jax.experimental import pallas as pl
from jax.experimental.pallas import tpu as pltpu

N = 4096
IN_FEAT = 64
OUT_FEAT = 64
BM = 512
H = N // 2


def _kern(featw_ref, w_ref, adj_ref, out_ref, x_ref):
    @pl.when(pl.program_id(0) == 0)
    def _():
        w = w_ref[...]
        x_ref[:H, :] = jnp.dot(
            featw_ref[:, :IN_FEAT], w, preferred_element_type=jnp.float32
        )
        x_ref[H:, :] = jnp.dot(
            featw_ref[:, IN_FEAT:], w, preferred_element_type=jnp.float32
        )

    out_ref[...] = jnp.dot(
        adj_ref[...], x_ref[...], preferred_element_type=jnp.float32
    )


@jax.jit
def kernel(feat, adj, weight2):
    featw = jnp.concatenate([feat[:H], feat[H:]], axis=1)
    grid = (N // BM,)
    return pl.pallas_call(
        _kern,
        grid=grid,
        in_specs=[
            pl.BlockSpec((H, 2 * IN_FEAT), lambda i: (0, 0)),
            pl.BlockSpec((IN_FEAT, OUT_FEAT), lambda i: (0, 0)),
            pl.BlockSpec((BM, N), lambda i: (i, 0)),
        ],
        out_specs=pl.BlockSpec((BM, OUT_FEAT), lambda i: (i, 0)),
        out_shape=jax.ShapeDtypeStruct((N, OUT_FEAT), jnp.float32),
        scratch_shapes=[pltpu.VMEM((N, OUT_FEAT), jnp.float32)],
    )(featw, weight2, adj)


# speedup vs baseline: 1.0171x; 1.0005x over previous
import jax
import jax.numpy as jnp
from jax.experimental import pallas as pl
from jax.experimental.pallas import tpu as pltpu

N = 4096
IN_FEAT = 64
OUT_FEAT = 64
BM = 512
H = N // 2
T = N // BM


def _kern(featw_ref, w_ref, adj_ref, out_hbm, x_ref, ybuf, osems):
    i = pl.program_id(0)

    @pl.when(i == 0)
    def _():
        w = w_ref[...]
        x_ref[:H, :] = jnp.dot(
            featw_ref[:, :IN_FEAT], w, preferred_element_type=jnp.float32
        )
        x_ref[H:, :] = jnp.dot(
            featw_ref[:, IN_FEAT:], w, preferred_element_type=jnp.float32
        )

    @pl.when(i >= 2)
    def _():
        pltpu.make_async_copy(
            ybuf.at[i % 2], out_hbm.at[pl.ds((i - 2) * BM, BM), :],
            osems.at[i % 2],
        ).wait()

    ybuf[i % 2] = jnp.dot(
        adj_ref[...], x_ref[...], preferred_element_type=jnp.float32
    )
    pltpu.make_async_copy(
        ybuf.at[i % 2], out_hbm.at[pl.ds(i * BM, BM), :], osems.at[i % 2]
    ).start()

    @pl.when(i == T - 1)
    def _():
        pltpu.make_async_copy(
            ybuf.at[(i - 1) % 2], out_hbm.at[pl.ds((i - 1) * BM, BM), :],
            osems.at[(i - 1) % 2],
        ).wait()
        pltpu.make_async_copy(
            ybuf.at[i % 2], out_hbm.at[pl.ds(i * BM, BM), :], osems.at[i % 2]
        ).wait()


@jax.jit
def kernel(feat, adj, weight2):
    featw = jnp.concatenate([feat[:H], feat[H:]], axis=1)
    grid = (N // BM,)
    return pl.pallas_call(
        _kern,
        grid=grid,
        in_specs=[
            pl.BlockSpec((H, 2 * IN_FEAT), lambda i: (0, 0)),
            pl.BlockSpec((IN_FEAT, OUT_FEAT), lambda i: (0, 0)),
            pl.BlockSpec((BM, N), lambda i: (i, 0)),
        ],
        out_specs=pl.BlockSpec(memory_space=pltpu.HBM),
        out_shape=jax.ShapeDtypeStruct((N, OUT_FEAT), jnp.float32),
        scratch_shapes=[
            pltpu.VMEM((N, OUT_FEAT), jnp.float32),
            pltpu.VMEM((2, BM, OUT_FEAT), jnp.float32),
            pltpu.SemaphoreType.DMA((2,)),
        ],
    )(featw, weight2, adj)
